# baseline (speedup 1.0000x reference)
import functools

import jax
import jax.numpy as jnp
from jax import lax
from jax.experimental import pallas as pl
from jax.experimental.pallas import tpu as pltpu
from jax.experimental.pallas import tpu_sc as plsc

B, T, V = 1024, 50, 1000
N = B * T
NC, NS = 2, 16
NW = NC * NS
NL = 16
PER_W = N // NW
C = 32
NCH = PER_W // C
VV = V * V
NOUT = 8
CHUNK_OUT = VV // NOUT
HR = 13
HW = 128


def _loss_body(emb_ref, c2_ref, out_ref):
    x = emb_ref[...]
    m = jnp.max(x, axis=1, keepdims=True)
    lse = m + jnp.log(jnp.sum(jnp.exp(x - m), axis=1, keepdims=True))
    w = c2_ref[0:V, :] + c2_ref[V:2 * V, :]
    out_ref[...] = (jnp.sum(w * (lse - x)) / N).reshape(1, 1)


def _sc_body(emb_hbm, idx_hbm, tgt_hbm, zeros_hbm,
             logits_hbm, c2_hbm,
             rows0, rows1, idx_all, tgt_all, idxv0, idxv1, fidx_v, ones_v,
             shared, gsem0, gsem1, osem0, osem1):
    cid = lax.axis_index("c")
    sid = lax.axis_index("s")
    wid = sid * NC + cid
    wbase = wid * PER_W

    @pl.when(sid == 0)
    def _():
        pltpu.sync_copy(zeros_hbm, shared)

    pltpu.sync_copy(idx_hbm.at[pl.ds(wbase, PER_W)], idx_all)
    pltpu.sync_copy(tgt_hbm.at[pl.ds(wbase, PER_W)], tgt_all)

    for k in range(HW // NL):
        ones_v[pl.ds(k * NL, NL)] = jnp.full((NL,), 1.0, jnp.float32)

    plsc.subcore_barrier()

    for j in range(HR):
        for k in range(HW // NL):
            flat = j * HW + k * NL
            sl = pl.ds(k * NL, NL)
            if flat + NL <= PER_W:
                fidx_v[sl] = (idx_all[pl.ds(flat, NL)] * V
                              + tgt_all[pl.ds(flat, NL)])
            else:
                fidx_v[sl] = jnp.full((NL,), 0, jnp.int32)
                ones_v[sl] = jnp.full((NL,), 0.0, jnp.float32)
        pltpu.sync_copy(ones_v, shared.at[fidx_v], add=True)

    rows = (rows0, rows1)
    idxv = (idxv0, idxv1)
    gsems = (gsem0, gsem1)
    osems = (osem0, osem1)

    def fill_fire(b, ci):
        for k in range(C // NL):
            sl = pl.ds(k * NL, NL)
            idxv[b][sl] = idx_all[pl.ds(ci * C + k * NL, NL)]
        pltpu.async_copy(emb_hbm.at[idxv[b]], rows[b], gsems[b])

    def gwait(b):
        pltpu.make_async_copy(emb_hbm.at[idxv[b]], rows[b], gsems[b]).wait()

    def out_fire(b, ci):
        pltpu.async_copy(rows[b], logits_hbm.at[pl.ds(wbase + ci * C, C)],
                         osems[b])

    def owait(b, ci):
        pltpu.make_async_copy(rows[b],
                              logits_hbm.at[pl.ds(wbase + ci * C, C)],
                              osems[b]).wait()

    fill_fire(0, 0)
    fill_fire(1, 1)

    def pair_body(i, carry):
        gwait(0)
        out_fire(0, i)
        gwait(1)
        out_fire(1, i + 1)

        @pl.when(i + 2 < NCH)
        def _():
            owait(0, i)
            fill_fire(0, i + 2)

        @pl.when(i + 3 < NCH)
        def _():
            owait(1, i + 1)
            fill_fire(1, i + 3)

        return carry

    lax.fori_loop(0, NCH // 2, lambda k, c: pair_body(k * 2, c), None)

    owait(0, NCH - 2)
    owait(1, NCH - 1)

    plsc.subcore_barrier()

    @pl.when(sid < NOUT)
    def _():
        pltpu.sync_copy(shared.at[pl.ds(sid * CHUNK_OUT, CHUNK_OUT)],
                        c2_hbm.at[cid, pl.ds(sid * CHUNK_OUT, CHUNK_OUT)])


def kernel(idx, targets, emb):
    mesh = plsc.VectorSubcoreMesh(core_axis_name="c", subcore_axis_name="s")
    sc = functools.partial(
        pl.kernel,
        mesh=mesh,
        compiler_params=pltpu.CompilerParams(use_tc_tiling_on_sc=False),
        out_type=(
            jax.ShapeDtypeStruct((N, V), jnp.float32),
            jax.ShapeDtypeStruct((NC, VV), jnp.float32),
        ),
        scratch_types=[
            pltpu.VMEM((C, V), jnp.float32),
            pltpu.VMEM((C, V), jnp.float32),
            pltpu.VMEM((PER_W,), jnp.int32),
            pltpu.VMEM((PER_W,), jnp.int32),
            pltpu.VMEM((C,), jnp.int32),
            pltpu.VMEM((C,), jnp.int32),
            pltpu.VMEM((HW,), jnp.int32),
            pltpu.VMEM((HW,), jnp.float32),
            pltpu.VMEM_SHARED((VV,), jnp.float32),
            pltpu.SemaphoreType.DMA,
            pltpu.SemaphoreType.DMA,
            pltpu.SemaphoreType.DMA,
            pltpu.SemaphoreType.DMA,
        ],
    )(_sc_body)
    zeros = jnp.zeros((VV,), jnp.float32)
    logits2, c2 = sc(emb, idx.reshape(-1), targets.reshape(-1), zeros)

    loss2 = pl.pallas_call(
        _loss_body,
        out_shape=jax.ShapeDtypeStruct((1, 1), jnp.float32),
    )(emb, c2.reshape(NC * V, V))
    return (logits2, loss2[0, 0])


# speedup vs baseline: 1.0060x; 1.0060x over previous
import functools

import jax
import jax.numpy as jnp
from jax import lax
from jax.experimental import pallas as pl
from jax.experimental.pallas import tpu as pltpu
from jax.experimental.pallas import tpu_sc as plsc

B, T, V = 1024, 50, 1000
N = B * T
NC, NS = 2, 16
NW = NC * NS
NL = 16
PER_W = N // NW
C = 32
NCH = PER_W // C
VV = V * V
NOUT = 8
CHUNK_OUT = VV // NOUT
HR = 13
HW = 128


def _loss_body(emb_ref, c2_ref, out_ref):
    x = emb_ref[...]
    m = jnp.max(x, axis=1, keepdims=True)
    lse = m + jnp.log(jnp.sum(jnp.exp(x - m), axis=1, keepdims=True))
    w = c2_ref[0:V, :] + c2_ref[V:2 * V, :]
    out_ref[...] = (jnp.sum(w * (lse - x)) / N).reshape(1, 1)


def _sc_body(emb_hbm, idx_hbm, tgt_hbm, zeros_hbm,
             logits_hbm, c2_hbm,
             rows0, rows1, idx_all, tgt_all, idxv0, idxv1, fidx_v, ones_v,
             shared, gsem0, gsem1, osem0, osem1):
    cid = lax.axis_index("c")
    sid = lax.axis_index("s")
    wid = sid * NC + cid
    wbase = wid * PER_W

    @pl.when(sid == 0)
    def _():
        pltpu.sync_copy(zeros_hbm, shared)

    pltpu.sync_copy(idx_hbm.at[pl.ds(wbase, PER_W)], idx_all)
    pltpu.sync_copy(tgt_hbm.at[pl.ds(wbase, PER_W)], tgt_all)

    for k in range(HW // NL):
        ones_v[pl.ds(k * NL, NL)] = jnp.full((NL,), 1.0, jnp.float32)

    plsc.subcore_barrier()

    for j in range(HR):
        for k in range(HW // NL):
            flat = j * HW + k * NL
            sl = pl.ds(k * NL, NL)
            if flat + NL <= PER_W:
                fidx_v[sl] = (idx_all[pl.ds(flat, NL)] * V
                              + tgt_all[pl.ds(flat, NL)])
            else:
                fidx_v[sl] = jnp.full((NL,), 0, jnp.int32)
                ones_v[sl] = jnp.full((NL,), 0.0, jnp.float32)
        pltpu.sync_copy(ones_v, shared.at[fidx_v], add=True)

    rows = (rows0, rows1)
    idxv = (idxv0, idxv1)
    gsems = (gsem0, gsem1)
    osems = (osem0, osem1)

    def fill_fire(b, ci):
        for k in range(C // NL):
            sl = pl.ds(k * NL, NL)
            idxv[b][sl] = idx_all[pl.ds(ci * C + k * NL, NL)]
        pltpu.async_copy(emb_hbm.at[idxv[b]], rows[b], gsems[b])

    def gwait(b):
        pltpu.make_async_copy(emb_hbm.at[idxv[b]], rows[b], gsems[b]).wait()

    def out_fire(b, ci):
        def row_copy(r, carry):
            pltpu.async_copy(
                rows[b].at[r],
                logits_hbm.at[pl.ds((wbase + ci * C + r) * V, V)],
                osems[b])
            return carry

        lax.fori_loop(0, C, row_copy, None)

    def owait(b, ci):
        pltpu.make_async_copy(emb_hbm.at[idxv[b]], rows[b], osems[b]).wait()

    fill_fire(0, 0)
    fill_fire(1, 1)

    def pair_body(i, carry):
        gwait(0)
        out_fire(0, i)
        gwait(1)
        out_fire(1, i + 1)

        @pl.when(i + 2 < NCH)
        def _():
            owait(0, i)
            fill_fire(0, i + 2)

        @pl.when(i + 3 < NCH)
        def _():
            owait(1, i + 1)
            fill_fire(1, i + 3)

        return carry

    lax.fori_loop(0, NCH // 2, lambda k, c: pair_body(k * 2, c), None)

    owait(0, NCH - 2)
    owait(1, NCH - 1)

    plsc.subcore_barrier()

    @pl.when(sid < NOUT)
    def _():
        pltpu.sync_copy(shared.at[pl.ds(sid * CHUNK_OUT, CHUNK_OUT)],
                        c2_hbm.at[cid, pl.ds(sid * CHUNK_OUT, CHUNK_OUT)])


def kernel(idx, targets, emb):
    mesh = plsc.VectorSubcoreMesh(core_axis_name="c", subcore_axis_name="s")
    sc = functools.partial(
        pl.kernel,
        mesh=mesh,
        compiler_params=pltpu.CompilerParams(use_tc_tiling_on_sc=False),
        out_type=(
            jax.ShapeDtypeStruct((N * V,), jnp.float32),
            jax.ShapeDtypeStruct((NC, VV), jnp.float32),
        ),
        scratch_types=[
            pltpu.VMEM((C, V), jnp.float32),
            pltpu.VMEM((C, V), jnp.float32),
            pltpu.VMEM((PER_W,), jnp.int32),
            pltpu.VMEM((PER_W,), jnp.int32),
            pltpu.VMEM((C,), jnp.int32),
            pltpu.VMEM((C,), jnp.int32),
            pltpu.VMEM((HW,), jnp.int32),
            pltpu.VMEM((HW,), jnp.float32),
            pltpu.VMEM_SHARED((VV,), jnp.float32),
            pltpu.SemaphoreType.DMA,
            pltpu.SemaphoreType.DMA,
            pltpu.SemaphoreType.DMA,
            pltpu.SemaphoreType.DMA,
        ],
    )(_sc_body)
    zeros = jnp.zeros((VV,), jnp.float32)
    logits_flat, c2 = sc(emb, idx.reshape(-1), targets.reshape(-1), zeros)
    logits2 = logits_flat.reshape(N, V)

    loss2 = pl.pallas_call(
        _loss_body,
        out_shape=jax.ShapeDtypeStruct((1, 1), jnp.float32),
    )(emb, c2.reshape(NC * V, V))
    return (logits2, loss2[0, 0])


# speedup vs baseline: 1.0684x; 1.0620x over previous
import functools

import jax
import jax.experimental.layout
import jax.numpy as jnp
from jax import lax
from jax.experimental import pallas as pl
from jax.experimental.pallas import tpu as pltpu
from jax.experimental.pallas import tpu_sc as plsc

B, T, V = 1024, 50, 1000
N = B * T
NC, NS = 2, 16
NW = NC * NS
NL = 16
PER_W = N // NW
C = 32
NCH = PER_W // C
VV = V * V
NOUT = 8
CHUNK_OUT = VV // NOUT
HR = 13
HW = 128


RB = 128
VP = 1024


def _relayout_body(flat_ref, out_ref):
    out_ref[...] = flat_ref[...].reshape(RB, VP)[:, :V].T


def _loss_body(emb_ref, c2_ref, out_ref):
    x = emb_ref[...]
    m = jnp.max(x, axis=1, keepdims=True)
    lse = m + jnp.log(jnp.sum(jnp.exp(x - m), axis=1, keepdims=True))
    w = c2_ref[0:V, :] + c2_ref[V:2 * V, :]
    out_ref[...] = (jnp.sum(w * (lse - x)) / N).reshape(1, 1)


def _sc_body(emb_hbm, idx_hbm, tgt_hbm, zeros_hbm,
             logits_hbm, c2_hbm,
             rows0, rows1, idx_all, tgt_all, idxv0, idxv1, fidx_v, ones_v,
             shared, gsem0, gsem1, osem0, osem1):
    cid = lax.axis_index("c")
    sid = lax.axis_index("s")
    wid = sid * NC + cid
    wbase = wid * PER_W

    @pl.when(sid == 0)
    def _():
        pltpu.sync_copy(zeros_hbm, shared)

    pltpu.sync_copy(idx_hbm.at[pl.ds(wbase, PER_W)], idx_all)
    pltpu.sync_copy(tgt_hbm.at[pl.ds(wbase, PER_W)], tgt_all)

    for k in range(HW // NL):
        ones_v[pl.ds(k * NL, NL)] = jnp.full((NL,), 1.0, jnp.float32)

    plsc.subcore_barrier()

    for j in range(HR):
        for k in range(HW // NL):
            flat = j * HW + k * NL
            sl = pl.ds(k * NL, NL)
            if flat + NL <= PER_W:
                fidx_v[sl] = (idx_all[pl.ds(flat, NL)] * V
                              + tgt_all[pl.ds(flat, NL)])
            else:
                fidx_v[sl] = jnp.full((NL,), 0, jnp.int32)
                ones_v[sl] = jnp.full((NL,), 0.0, jnp.float32)
        pltpu.sync_copy(ones_v, shared.at[fidx_v], add=True)

    rows = (rows0, rows1)
    idxv = (idxv0, idxv1)
    gsems = (gsem0, gsem1)
    osems = (osem0, osem1)

    def fill_fire(b, ci):
        for k in range(C // NL):
            sl = pl.ds(k * NL, NL)
            idxv[b][sl] = idx_all[pl.ds(ci * C + k * NL, NL)]
        pltpu.async_copy(emb_hbm.at[idxv[b]], rows[b], gsems[b])

    def gwait(b):
        pltpu.make_async_copy(emb_hbm.at[idxv[b]], rows[b], gsems[b]).wait()

    def out_fire(b, ci):
        def row_copy(r, carry):
            pltpu.async_copy(
                rows[b].at[r],
                logits_hbm.at[pl.ds((wbase + ci * C + r) * VP, V)],
                osems[b])
            return carry

        lax.fori_loop(0, C, row_copy, None)

    def owait(b, ci):
        pltpu.make_async_copy(emb_hbm.at[idxv[b]], rows[b], osems[b]).wait()

    fill_fire(0, 0)
    fill_fire(1, 1)

    def pair_body(i, carry):
        gwait(0)
        out_fire(0, i)
        gwait(1)
        out_fire(1, i + 1)

        @pl.when(i + 2 < NCH)
        def _():
            owait(0, i)
            fill_fire(0, i + 2)

        @pl.when(i + 3 < NCH)
        def _():
            owait(1, i + 1)
            fill_fire(1, i + 3)

        return carry

    lax.fori_loop(0, NCH // 2, lambda k, c: pair_body(k * 2, c), None)

    owait(0, NCH - 2)
    owait(1, NCH - 1)

    plsc.subcore_barrier()

    @pl.when(sid < NOUT)
    def _():
        pltpu.sync_copy(shared.at[pl.ds(sid * CHUNK_OUT, CHUNK_OUT)],
                        c2_hbm.at[cid, pl.ds(sid * CHUNK_OUT, CHUNK_OUT)])


def kernel(idx, targets, emb):
    mesh = plsc.VectorSubcoreMesh(core_axis_name="c", subcore_axis_name="s")
    sc = functools.partial(
        pl.kernel,
        mesh=mesh,
        compiler_params=pltpu.CompilerParams(use_tc_tiling_on_sc=False),
        out_type=(
            jax.ShapeDtypeStruct((N * VP,), jnp.float32),
            jax.ShapeDtypeStruct((NC, VV), jnp.float32),
        ),
        scratch_types=[
            pltpu.VMEM((C, V), jnp.float32),
            pltpu.VMEM((C, V), jnp.float32),
            pltpu.VMEM((PER_W,), jnp.int32),
            pltpu.VMEM((PER_W,), jnp.int32),
            pltpu.VMEM((C,), jnp.int32),
            pltpu.VMEM((C,), jnp.int32),
            pltpu.VMEM((HW,), jnp.int32),
            pltpu.VMEM((HW,), jnp.float32),
            pltpu.VMEM_SHARED((VV,), jnp.float32),
            pltpu.SemaphoreType.DMA,
            pltpu.SemaphoreType.DMA,
            pltpu.SemaphoreType.DMA,
            pltpu.SemaphoreType.DMA,
        ],
    )(_sc_body)
    zeros = jnp.zeros((VV,), jnp.float32)
    logits_flat, c2 = sc(emb, idx.reshape(-1), targets.reshape(-1), zeros)

    logits_t = pl.pallas_call(
        _relayout_body,
        grid=(N // RB,),
        in_specs=[pl.BlockSpec((RB * VP,), lambda i: (i,))],
        out_specs=pl.BlockSpec((V, RB), lambda i: (0, i)),
        out_shape=jax.ShapeDtypeStruct((V, N), jnp.float32),
    )(logits_flat)
    logits2 = logits_t.T

    loss2 = pl.pallas_call(
        _loss_body,
        out_shape=jax.ShapeDtypeStruct((1, 1), jnp.float32),
    )(emb, c2.reshape(NC * V, V))
    return (logits2, loss2[0, 0])


# speedup vs baseline: 1.5251x; 1.4274x over previous
import functools

import jax
import jax.experimental.layout
import jax.numpy as jnp
from jax import lax
from jax.experimental import pallas as pl
from jax.experimental.pallas import tpu as pltpu
from jax.experimental.pallas import tpu_sc as plsc

B, T, V = 1024, 50, 1000
N = B * T
NC, NS = 2, 16
NW = NC * NS
NL = 16
PER_W = N // NW
C = 32
NCH = PER_W // C
VV = V * V
NOUT = 8
CHUNK_OUT = VV // NOUT
HR = 13
HW = 128


RB = 512
VP = 1024


def _relayout_body(flat_ref, out_ref):
    out_ref[...] = flat_ref[...].reshape(RB, VP).T[:V, :]


def _loss_body(emb_ref, c2_ref, out_ref):
    x = emb_ref[...]
    m = jnp.max(x, axis=1, keepdims=True)
    lse = m + jnp.log(jnp.sum(jnp.exp(x - m), axis=1, keepdims=True))
    w = c2_ref[0:V, :] + c2_ref[V:2 * V, :]
    out_ref[...] = (jnp.sum(w * (lse - x)) / N).reshape(1, 1)


def _sc_body(emb_hbm, idx_hbm, tgt_hbm, zeros_hbm,
             logits_hbm, c2_hbm,
             rows0, rows1, idx_all, tgt_all, idxv0, idxv1, fidx_v, ones_v,
             shared, gsem0, gsem1, osem0, osem1):
    cid = lax.axis_index("c")
    sid = lax.axis_index("s")
    wid = sid * NC + cid
    wbase = wid * PER_W

    @pl.when(sid == 0)
    def _():
        pltpu.sync_copy(zeros_hbm, shared)

    pltpu.sync_copy(idx_hbm.at[pl.ds(wbase, PER_W)], idx_all)
    pltpu.sync_copy(tgt_hbm.at[pl.ds(wbase, PER_W)], tgt_all)

    for k in range(HW // NL):
        ones_v[pl.ds(k * NL, NL)] = jnp.full((NL,), 1.0, jnp.float32)

    plsc.subcore_barrier()

    for j in range(HR):
        for k in range(HW // NL):
            flat = j * HW + k * NL
            sl = pl.ds(k * NL, NL)
            if flat + NL <= PER_W:
                fidx_v[sl] = (idx_all[pl.ds(flat, NL)] * V
                              + tgt_all[pl.ds(flat, NL)])
            else:
                fidx_v[sl] = jnp.full((NL,), 0, jnp.int32)
                ones_v[sl] = jnp.full((NL,), 0.0, jnp.float32)
        pltpu.sync_copy(ones_v, shared.at[fidx_v], add=True)

    rows = (rows0, rows1)
    idxv = (idxv0, idxv1)
    gsems = (gsem0, gsem1)
    osems = (osem0, osem1)

    def fill_fire(b, ci):
        for k in range(C // NL):
            sl = pl.ds(k * NL, NL)
            idxv[b][sl] = idx_all[pl.ds(ci * C + k * NL, NL)]
        pltpu.async_copy(emb_hbm.at[idxv[b]], rows[b], gsems[b])

    def gwait(b):
        pltpu.make_async_copy(emb_hbm.at[idxv[b]], rows[b], gsems[b]).wait()

    def out_fire(b, ci):
        def row_copy(r, carry):
            pltpu.async_copy(
                rows[b].at[r],
                logits_hbm.at[pl.ds((wbase + ci * C + r) * VP, V)],
                osems[b])
            return carry

        lax.fori_loop(0, C, row_copy, None)

    def owait(b, ci):
        pltpu.make_async_copy(emb_hbm.at[idxv[b]], rows[b], osems[b]).wait()

    fill_fire(0, 0)
    fill_fire(1, 1)

    def pair_body(i, carry):
        gwait(0)
        out_fire(0, i)
        gwait(1)
        out_fire(1, i + 1)

        @pl.when(i + 2 < NCH)
        def _():
            owait(0, i)
            fill_fire(0, i + 2)

        @pl.when(i + 3 < NCH)
        def _():
            owait(1, i + 1)
            fill_fire(1, i + 3)

        return carry

    lax.fori_loop(0, NCH // 2, lambda k, c: pair_body(k * 2, c), None)

    owait(0, NCH - 2)
    owait(1, NCH - 1)

    plsc.subcore_barrier()

    @pl.when(sid < NOUT)
    def _():
        pltpu.sync_copy(shared.at[pl.ds(sid * CHUNK_OUT, CHUNK_OUT)],
                        c2_hbm.at[cid, pl.ds(sid * CHUNK_OUT, CHUNK_OUT)])


def kernel(idx, targets, emb):
    mesh = plsc.VectorSubcoreMesh(core_axis_name="c", subcore_axis_name="s")
    sc = functools.partial(
        pl.kernel,
        mesh=mesh,
        compiler_params=pltpu.CompilerParams(use_tc_tiling_on_sc=False),
        out_type=(
            jax.ShapeDtypeStruct((N * VP,), jnp.float32),
            jax.ShapeDtypeStruct((NC, VV), jnp.float32),
        ),
        scratch_types=[
            pltpu.VMEM((C, V), jnp.float32),
            pltpu.VMEM((C, V), jnp.float32),
            pltpu.VMEM((PER_W,), jnp.int32),
            pltpu.VMEM((PER_W,), jnp.int32),
            pltpu.VMEM((C,), jnp.int32),
            pltpu.VMEM((C,), jnp.int32),
            pltpu.VMEM((HW,), jnp.int32),
            pltpu.VMEM((HW,), jnp.float32),
            pltpu.VMEM_SHARED((VV,), jnp.float32),
            pltpu.SemaphoreType.DMA,
            pltpu.SemaphoreType.DMA,
            pltpu.SemaphoreType.DMA,
            pltpu.SemaphoreType.DMA,
        ],
    )(_sc_body)
    zeros = jnp.zeros((VV,), jnp.float32)
    logits_flat, c2 = sc(emb, idx.reshape(-1), targets.reshape(-1), zeros)

    logits_t = pl.pallas_call(
        _relayout_body,
        grid=(N // RB,),
        in_specs=[pl.BlockSpec((RB * VP,), lambda i: (i,))],
        out_specs=pl.BlockSpec((V, RB), lambda i: (0, i)),
        out_shape=jax.ShapeDtypeStruct((V, N), jnp.float32),
    )(logits_flat)
    logits2 = logits_t.T

    loss2 = pl.pallas_call(
        _loss_body,
        out_shape=jax.ShapeDtypeStruct((1, 1), jnp.float32),
    )(emb, c2.reshape(NC * V, V))
    return (logits2, loss2[0, 0])


# speedup vs baseline: 1.6546x; 1.0849x over previous
import functools

import jax
import jax.experimental.layout
import jax.numpy as jnp
from jax import lax
from jax.experimental import pallas as pl
from jax.experimental.pallas import tpu as pltpu
from jax.experimental.pallas import tpu_sc as plsc

B, T, V = 1024, 50, 1000
N = B * T
NC, NS = 2, 16
NW = NC * NS
NL = 16
PER_W = N // NW
C = 32
NCH = PER_W // C
VV = V * V
NOUT = 8
CHUNK_OUT = VV // NOUT
HR = 13
HW = 128


RB = 1024
VP = 1024


def _relayout_body(flat_ref, out_ref):
    out_ref[...] = flat_ref[...].reshape(RB, VP).T[:V, :]


def _loss_body(emb_ref, c2_ref, out_ref):
    x = emb_ref[...]
    m = jnp.max(x, axis=1, keepdims=True)
    lse = m + jnp.log(jnp.sum(jnp.exp(x - m), axis=1, keepdims=True))
    w = c2_ref[0:V, :] + c2_ref[V:2 * V, :]
    out_ref[...] = (jnp.sum(w * (lse - x)) / N).reshape(1, 1)


def _sc_body(emb_hbm, idx_hbm, tgt_hbm, zeros_hbm,
             logits_hbm, c2_hbm,
             rows0, rows1, idx_all, tgt_all, idxv0, idxv1, fidx_v, ones_v,
             shared, gsem0, gsem1, osem0, osem1):
    cid = lax.axis_index("c")
    sid = lax.axis_index("s")
    wid = sid * NC + cid
    wbase = wid * PER_W

    @pl.when(sid == 0)
    def _():
        pltpu.sync_copy(zeros_hbm, shared)

    pltpu.sync_copy(idx_hbm.at[pl.ds(wbase, PER_W)], idx_all)
    pltpu.sync_copy(tgt_hbm.at[pl.ds(wbase, PER_W)], tgt_all)

    for k in range(HW // NL):
        ones_v[pl.ds(k * NL, NL)] = jnp.full((NL,), 1.0, jnp.float32)

    plsc.subcore_barrier()

    for j in range(HR):
        for k in range(HW // NL):
            flat = j * HW + k * NL
            sl = pl.ds(k * NL, NL)
            if flat + NL <= PER_W:
                fidx_v[sl] = (idx_all[pl.ds(flat, NL)] * V
                              + tgt_all[pl.ds(flat, NL)])
            else:
                fidx_v[sl] = jnp.full((NL,), 0, jnp.int32)
                ones_v[sl] = jnp.full((NL,), 0.0, jnp.float32)
        pltpu.sync_copy(ones_v, shared.at[fidx_v], add=True)

    rows = (rows0, rows1)
    idxv = (idxv0, idxv1)
    gsems = (gsem0, gsem1)
    osems = (osem0, osem1)

    def fill_fire(b, ci):
        for k in range(C // NL):
            sl = pl.ds(k * NL, NL)
            idxv[b][sl] = idx_all[pl.ds(ci * C + k * NL, NL)]
        pltpu.async_copy(emb_hbm.at[idxv[b]], rows[b], gsems[b])

    def gwait(b):
        pltpu.make_async_copy(emb_hbm.at[idxv[b]], rows[b], gsems[b]).wait()

    def out_fire(b, ci):
        def row_copy(r, carry):
            pltpu.async_copy(
                rows[b].at[r],
                logits_hbm.at[pl.ds((wbase + ci * C + r) * VP, V)],
                osems[b])
            return carry

        lax.fori_loop(0, C, row_copy, None)

    def owait(b, ci):
        pltpu.make_async_copy(emb_hbm.at[idxv[b]], rows[b], osems[b]).wait()

    fill_fire(0, 0)
    fill_fire(1, 1)

    def pair_body(i, carry):
        gwait(0)
        out_fire(0, i)
        gwait(1)
        out_fire(1, i + 1)

        @pl.when(i + 2 < NCH)
        def _():
            owait(0, i)
            fill_fire(0, i + 2)

        @pl.when(i + 3 < NCH)
        def _():
            owait(1, i + 1)
            fill_fire(1, i + 3)

        return carry

    lax.fori_loop(0, NCH // 2, lambda k, c: pair_body(k * 2, c), None)

    owait(0, NCH - 2)
    owait(1, NCH - 1)

    plsc.subcore_barrier()

    @pl.when(sid < NOUT)
    def _():
        pltpu.sync_copy(shared.at[pl.ds(sid * CHUNK_OUT, CHUNK_OUT)],
                        c2_hbm.at[cid, pl.ds(sid * CHUNK_OUT, CHUNK_OUT)])


def kernel(idx, targets, emb):
    mesh = plsc.VectorSubcoreMesh(core_axis_name="c", subcore_axis_name="s")
    sc = functools.partial(
        pl.kernel,
        mesh=mesh,
        compiler_params=pltpu.CompilerParams(use_tc_tiling_on_sc=False),
        out_type=(
            jax.ShapeDtypeStruct((N * VP,), jnp.float32),
            jax.ShapeDtypeStruct((NC, VV), jnp.float32),
        ),
        scratch_types=[
            pltpu.VMEM((C, V), jnp.float32),
            pltpu.VMEM((C, V), jnp.float32),
            pltpu.VMEM((PER_W,), jnp.int32),
            pltpu.VMEM((PER_W,), jnp.int32),
            pltpu.VMEM((C,), jnp.int32),
            pltpu.VMEM((C,), jnp.int32),
            pltpu.VMEM((HW,), jnp.int32),
            pltpu.VMEM((HW,), jnp.float32),
            pltpu.VMEM_SHARED((VV,), jnp.float32),
            pltpu.SemaphoreType.DMA,
            pltpu.SemaphoreType.DMA,
            pltpu.SemaphoreType.DMA,
            pltpu.SemaphoreType.DMA,
        ],
    )(_sc_body)
    zeros = jnp.zeros((VV,), jnp.float32)
    logits_flat, c2 = sc(emb, idx.reshape(-1), targets.reshape(-1), zeros)

    logits_t = pl.pallas_call(
        _relayout_body,
        grid=(N // RB,),
        in_specs=[pl.BlockSpec((RB * VP,), lambda i: (i,))],
        out_specs=pl.BlockSpec((V, RB), lambda i: (0, i)),
        out_shape=jax.ShapeDtypeStruct((V, N), jnp.float32),
    )(logits_flat)
    logits2 = logits_t.T

    loss2 = pl.pallas_call(
        _loss_body,
        out_shape=jax.ShapeDtypeStruct((1, 1), jnp.float32),
    )(emb, c2.reshape(NC * V, V))
    return (logits2, loss2[0, 0])


# speedup vs baseline: 1.6568x; 1.0013x over previous
import functools

import jax
import jax.experimental.layout
import jax.numpy as jnp
from jax import lax
from jax.experimental import pallas as pl
from jax.experimental.pallas import tpu as pltpu
from jax.experimental.pallas import tpu_sc as plsc

B, T, V = 1024, 50, 1000
N = B * T
NC, NS = 2, 16
NW = NC * NS
NL = 16
PER_W = N // NW
C = 32
NCH = PER_W // C
VV = V * V
NOUT = 8
CHUNK_OUT = VV // NOUT
HR = 13
HW = 128


RB = 2048
VP = 1024


def _relayout_body(flat_ref, out_ref):
    out_ref[...] = flat_ref[...].reshape(RB, VP).T[:V, :]


def _loss_body(emb_ref, c2_ref, out_ref):
    x = emb_ref[...]
    m = jnp.max(x, axis=1, keepdims=True)
    lse = m + jnp.log(jnp.sum(jnp.exp(x - m), axis=1, keepdims=True))
    w = c2_ref[0:V, :] + c2_ref[V:2 * V, :]
    out_ref[...] = (jnp.sum(w * (lse - x)) / N).reshape(1, 1)


def _sc_body(emb_hbm, idx_hbm, tgt_hbm, zeros_hbm,
             logits_hbm, c2_hbm,
             rows0, rows1, idx_all, tgt_all, idxv0, idxv1, fidx_v, ones_v,
             shared, gsem0, gsem1, osem0, osem1):
    cid = lax.axis_index("c")
    sid = lax.axis_index("s")
    wid = sid * NC + cid
    wbase = wid * PER_W

    @pl.when(sid == 0)
    def _():
        pltpu.sync_copy(zeros_hbm, shared)

    pltpu.sync_copy(idx_hbm.at[pl.ds(wbase, PER_W)], idx_all)
    pltpu.sync_copy(tgt_hbm.at[pl.ds(wbase, PER_W)], tgt_all)

    for k in range(HW // NL):
        ones_v[pl.ds(k * NL, NL)] = jnp.full((NL,), 1.0, jnp.float32)

    plsc.subcore_barrier()

    for j in range(HR):
        for k in range(HW // NL):
            flat = j * HW + k * NL
            sl = pl.ds(k * NL, NL)
            if flat + NL <= PER_W:
                fidx_v[sl] = (idx_all[pl.ds(flat, NL)] * V
                              + tgt_all[pl.ds(flat, NL)])
            else:
                fidx_v[sl] = jnp.full((NL,), 0, jnp.int32)
                ones_v[sl] = jnp.full((NL,), 0.0, jnp.float32)
        pltpu.sync_copy(ones_v, shared.at[fidx_v], add=True)

    rows = (rows0, rows1)
    idxv = (idxv0, idxv1)
    gsems = (gsem0, gsem1)
    osems = (osem0, osem1)

    def fill_fire(b, ci):
        for k in range(C // NL):
            sl = pl.ds(k * NL, NL)
            idxv[b][sl] = idx_all[pl.ds(ci * C + k * NL, NL)]
        pltpu.async_copy(emb_hbm.at[idxv[b]], rows[b], gsems[b])

    def gwait(b):
        pltpu.make_async_copy(emb_hbm.at[idxv[b]], rows[b], gsems[b]).wait()

    def out_fire(b, ci):
        def row_copy(r, carry):
            pltpu.async_copy(
                rows[b].at[r],
                logits_hbm.at[pl.ds((wbase + ci * C + r) * VP, V)],
                osems[b])
            return carry

        lax.fori_loop(0, C, row_copy, None)

    def owait(b, ci):
        pltpu.make_async_copy(emb_hbm.at[idxv[b]], rows[b], osems[b]).wait()

    fill_fire(0, 0)
    fill_fire(1, 1)

    def pair_body(i, carry):
        gwait(0)
        out_fire(0, i)
        gwait(1)
        out_fire(1, i + 1)

        @pl.when(i + 2 < NCH)
        def _():
            owait(0, i)
            fill_fire(0, i + 2)

        @pl.when(i + 3 < NCH)
        def _():
            owait(1, i + 1)
            fill_fire(1, i + 3)

        return carry

    lax.fori_loop(0, NCH // 2, lambda k, c: pair_body(k * 2, c), None)

    owait(0, NCH - 2)
    owait(1, NCH - 1)

    plsc.subcore_barrier()

    @pl.when(sid < NOUT)
    def _():
        pltpu.sync_copy(shared.at[pl.ds(sid * CHUNK_OUT, CHUNK_OUT)],
                        c2_hbm.at[cid, pl.ds(sid * CHUNK_OUT, CHUNK_OUT)])


def kernel(idx, targets, emb):
    mesh = plsc.VectorSubcoreMesh(core_axis_name="c", subcore_axis_name="s")
    sc = functools.partial(
        pl.kernel,
        mesh=mesh,
        compiler_params=pltpu.CompilerParams(use_tc_tiling_on_sc=False),
        out_type=(
            jax.ShapeDtypeStruct((N * VP,), jnp.float32),
            jax.ShapeDtypeStruct((NC, VV), jnp.float32),
        ),
        scratch_types=[
            pltpu.VMEM((C, V), jnp.float32),
            pltpu.VMEM((C, V), jnp.float32),
            pltpu.VMEM((PER_W,), jnp.int32),
            pltpu.VMEM((PER_W,), jnp.int32),
            pltpu.VMEM((C,), jnp.int32),
            pltpu.VMEM((C,), jnp.int32),
            pltpu.VMEM((HW,), jnp.int32),
            pltpu.VMEM((HW,), jnp.float32),
            pltpu.VMEM_SHARED((VV,), jnp.float32),
            pltpu.SemaphoreType.DMA,
            pltpu.SemaphoreType.DMA,
            pltpu.SemaphoreType.DMA,
            pltpu.SemaphoreType.DMA,
        ],
    )(_sc_body)
    zeros = jnp.zeros((VV,), jnp.float32)
    logits_flat, c2 = sc(emb, idx.reshape(-1), targets.reshape(-1), zeros)

    logits_t = pl.pallas_call(
        _relayout_body,
        grid=(N // RB,),
        in_specs=[pl.BlockSpec((RB * VP,), lambda i: (i,))],
        out_specs=pl.BlockSpec((V, RB), lambda i: (0, i)),
        out_shape=jax.ShapeDtypeStruct((V, N), jnp.float32),
    )(logits_flat)
    logits2 = logits_t.T

    loss2 = pl.pallas_call(
        _loss_body,
        out_shape=jax.ShapeDtypeStruct((1, 1), jnp.float32),
    )(emb, c2.reshape(NC * V, V))
    return (logits2, loss2[0, 0])


# speedup vs baseline: 1.6620x; 1.0031x over previous
import functools

import jax
import jax.numpy as jnp
from jax import lax
from jax.experimental import pallas as pl
from jax.experimental.pallas import tpu as pltpu
from jax.experimental.pallas import tpu_sc as plsc

B, T, V = 1024, 50, 1000
N = B * T
NC, NS = 2, 16
NW = NC * NS
NL = 16
PER_W = N // NW
C = 32
NCH = PER_W // C
VV = V * V
NOUT = 8
CHUNK_OUT = VV // NOUT
HR = 13
HW = 128


RB = 2048
VP = 1024


def _relayout_body(flat_ref, out_ref):
    out_ref[...] = flat_ref[...].reshape(RB, VP).T[:V, :]


def _loss_body(emb_ref, c2_ref, out_ref):
    x = emb_ref[...]
    m = jnp.max(x, axis=1, keepdims=True)
    lse = m + jnp.log(jnp.sum(jnp.exp(x - m), axis=1, keepdims=True))
    w = c2_ref[0:V, :] + c2_ref[V:2 * V, :]
    out_ref[...] = (jnp.sum(w * (lse - x)) / N).reshape(1, 1)


def _sc_body(emb_hbm, idx_hbm, tgt_hbm, zeros_hbm,
             logits_hbm, c2_hbm,
             rows0, rows1, idx_all, tgt_all, idxv0, idxv1, fidx_v, ones_v,
             shared, gsem0, gsem1, osem0, osem1):
    cid = lax.axis_index("c")
    sid = lax.axis_index("s")
    wid = sid * NC + cid
    wbase = wid * PER_W

    @pl.when(sid == 0)
    def _():
        pltpu.sync_copy(zeros_hbm, shared)

    pltpu.sync_copy(idx_hbm.at[pl.ds(wbase, PER_W)], idx_all)
    pltpu.sync_copy(tgt_hbm.at[pl.ds(wbase, PER_W)], tgt_all)

    for k in range(HW // NL):
        ones_v[pl.ds(k * NL, NL)] = jnp.full((NL,), 1.0, jnp.float32)

    plsc.subcore_barrier()

    for j in range(HR):
        for k in range(HW // NL):
            flat = j * HW + k * NL
            sl = pl.ds(k * NL, NL)
            if flat + NL <= PER_W:
                fidx_v[sl] = (idx_all[pl.ds(flat, NL)] * V
                              + tgt_all[pl.ds(flat, NL)])
            else:
                fidx_v[sl] = jnp.full((NL,), 0, jnp.int32)
                ones_v[sl] = jnp.full((NL,), 0.0, jnp.float32)
        pltpu.sync_copy(ones_v, shared.at[fidx_v], add=True)

    rows = (rows0, rows1)
    idxv = (idxv0, idxv1)
    gsems = (gsem0, gsem1)
    osems = (osem0, osem1)

    def fill_fire(b, ci):
        for k in range(C // NL):
            sl = pl.ds(k * NL, NL)
            idxv[b][sl] = idx_all[pl.ds(ci * C + k * NL, NL)]
        pltpu.async_copy(emb_hbm.at[idxv[b]], rows[b], gsems[b])

    def gwait(b):
        pltpu.make_async_copy(emb_hbm.at[idxv[b]], rows[b], gsems[b]).wait()

    def out_fire(b, ci):
        def row_copy(r, carry):
            pltpu.async_copy(
                rows[b].at[r],
                logits_hbm.at[pl.ds((wbase + ci * C + r) * VP, V)],
                osems[b])
            return carry

        lax.fori_loop(0, C, row_copy, None)

    def owait(b, ci):
        pltpu.make_async_copy(emb_hbm.at[idxv[b]], rows[b], osems[b]).wait()

    fill_fire(0, 0)
    fill_fire(1, 1)

    def pair_body(i, carry):
        gwait(0)
        out_fire(0, i)
        gwait(1)
        out_fire(1, i + 1)

        @pl.when(i + 2 < NCH)
        def _():
            owait(0, i)
            fill_fire(0, i + 2)

        @pl.when(i + 3 < NCH)
        def _():
            owait(1, i + 1)
            fill_fire(1, i + 3)

        return carry

    lax.fori_loop(0, NCH // 2, lambda k, c: pair_body(k * 2, c), None)

    owait(0, NCH - 2)
    owait(1, NCH - 1)

    plsc.subcore_barrier()

    @pl.when(sid < NOUT)
    def _():
        pltpu.sync_copy(shared.at[pl.ds(sid * CHUNK_OUT, CHUNK_OUT)],
                        c2_hbm.at[cid, pl.ds(sid * CHUNK_OUT, CHUNK_OUT)])


def kernel(idx, targets, emb):
    mesh = plsc.VectorSubcoreMesh(core_axis_name="c", subcore_axis_name="s")
    sc = functools.partial(
        pl.kernel,
        mesh=mesh,
        compiler_params=pltpu.CompilerParams(use_tc_tiling_on_sc=False),
        out_type=(
            jax.ShapeDtypeStruct((N * VP,), jnp.float32),
            jax.ShapeDtypeStruct((NC, VV), jnp.float32),
        ),
        scratch_types=[
            pltpu.VMEM((C, V), jnp.float32),
            pltpu.VMEM((C, V), jnp.float32),
            pltpu.VMEM((PER_W,), jnp.int32),
            pltpu.VMEM((PER_W,), jnp.int32),
            pltpu.VMEM((C,), jnp.int32),
            pltpu.VMEM((C,), jnp.int32),
            pltpu.VMEM((HW,), jnp.int32),
            pltpu.VMEM((HW,), jnp.float32),
            pltpu.VMEM_SHARED((VV,), jnp.float32),
            pltpu.SemaphoreType.DMA,
            pltpu.SemaphoreType.DMA,
            pltpu.SemaphoreType.DMA,
            pltpu.SemaphoreType.DMA,
        ],
    )(_sc_body)
    zeros = jnp.zeros((VV,), jnp.float32)
    logits_flat, c2 = sc(emb, idx.reshape(-1), targets.reshape(-1), zeros)

    logits_t = pl.pallas_call(
        _relayout_body,
        grid=(N // RB,),
        in_specs=[pl.BlockSpec((RB * VP,), lambda i: (i,))],
        out_specs=pl.BlockSpec((V, RB), lambda i: (0, i)),
        out_shape=jax.ShapeDtypeStruct((V, N), jnp.float32),
    )(logits_flat)
    logits2 = logits_t.T

    loss2 = pl.pallas_call(
        _loss_body,
        out_shape=jax.ShapeDtypeStruct((1, 1), jnp.float32),
    )(emb, c2.reshape(NC * V, V))
    return (logits2, loss2[0, 0])
